# baseline (speedup 1.0000x reference)
import functools

import jax
import jax.numpy as jnp
from jax import lax
from jax.experimental import pallas as pl
from jax.experimental.pallas import tpu as pltpu
from jax.experimental.pallas import tpu_sc as plsc

NC = 2
NS = 16
L = 16
NW = NC * NS

NROWS = 1000000
BATCH_ = 16384
EMB_ = 64
BPW = BATCH_ // NW
NCOL = (NROWS + 127) // 128
BW = 256
BWS = BW.bit_length() - 1
NRING = 4
NCB = (NCOL * 128 + BW - 1) // BW
CBW = 123
MAXOFF = NCOL * 128 - BW
SLOTS = 8
NDROWS = BATCH_ + NW


def _extract_body(ids_hbm, tblT, rows_out,
                  allids, cnt, cand_id, cand_pos, ring, rowring,
                  sem_s0, sem_s1, sem_s2, sem_s3, sem_w):
    wid = lax.axis_index("s") * NC + lax.axis_index("c")
    base_blk = wid * CBW
    myblocks = jnp.clip(NCB - base_blk, 0, CBW)

    pltpu.sync_copy(ids_hbm, allids)

    lane = lax.iota(jnp.int32, L)
    zeros = jnp.zeros((L,), jnp.int32)
    ones = jnp.full((L,), 1, jnp.int32)

    def zcnt(i, _):
        cnt[pl.ds(i * L, L)] = zeros
        return 0
    lax.fori_loop(0, CBW + 1, zcnt, 0)

    def filt(c, _):
        v = allids[pl.ds(c * L, L)]
        blk = lax.shift_right_logical(v, BWS) - base_blk
        mask = (blk >= 0) & (blk < myblocks)
        b = jnp.clip(blk, 0, CBW - 1)
        key = b * L + lane
        o = jnp.minimum(plsc.load_gather(cnt, [key]), SLOTS - 1)
        flat = (b * SLOTS + o) * L + lane
        plsc.store_scatter(cand_id, [flat], v, mask=mask)
        plsc.store_scatter(cand_pos, [flat], c * L + lane, mask=mask)
        plsc.addupdate_scatter(cnt, [key], ones, mask=mask)
        return 0
    lax.fori_loop(0, BATCH_ // L, filt, 0)

    def blk_off(t):
        return jnp.minimum((base_blk + t) * BW, MAXOFF)

    sems = (sem_s0, sem_s1, sem_s2, sem_s3)

    def fire_block(t, b):
        pltpu.async_copy(tblT.at[:, pl.ds(blk_off(t), BW)],
                         ring.at[b], sems[b])

    for b in range(NRING):
        @pl.when(myblocks > b)
        def _(b=b):
            fire_block(b, b)

    def drain_rows(i, _):
        pltpu.make_async_copy(rowring.at[0], rows_out.at[pl.ds(0, L)],
                              sem_w).wait()
        return 0

    def bcast(vec, k):
        return lax.gather(
            vec, jnp.full((L, 1), k, jnp.int32),
            lax.GatherDimensionNumbers(
                offset_dims=(), collapsed_slice_dims=(0,),
                start_index_map=(0,)),
            (1,), mode=lax.GatherScatterMode.PROMISE_IN_BOUNDS)

    def scan_one(t, hcnt, b):
        cvec = cnt[pl.ds(t * L, L)]
        jmax = jnp.max(cvec)
        slot = jnp.full((L,), b, jnp.int32)
        off = blk_off(t)

        def scan_slot(j, hcnt):
            row = cand_id[pl.ds((t * SLOTS + j) * L, L)]
            pvec = cand_pos[pl.ds((t * SLOTS + j) * L, L)]
            hit = cvec > j
            key = jnp.where(hit, zeros, ones)
            _, ids_s = plsc.sort_key_val(key, row)
            _, pos_s = plsc.sort_key_val(key, pvec)
            nh = jnp.sum(jnp.where(hit, ones, zeros))
            cv_all = jnp.clip(ids_s - off, 0, BW - 1)
            hs = hcnt & 3

            def perhit(k, _):
                ck = bcast(cv_all, k)
                for j4 in range(EMB_ // L):
                    dv = j4 * L + lane
                    vals = plsc.load_gather(ring, [slot, dv, ck])
                    rowring[hs, k, pl.ds(j4 * L, L)] = vals
                return 0
            lax.fori_loop(0, nh, perhit, 0)

            posv = jnp.where(lane < nh, pos_s, BATCH_ + wid)
            pltpu.async_copy(rowring.at[hs], rows_out.at[posv], sem_w)
            @pl.when((hcnt & 3) == 3)
            def _():
                lax.fori_loop(0, 4, drain_rows, 0)

            return hcnt + 1

        return lax.fori_loop(0, jmax, scan_slot, hcnt)

    def scan_quad(t4, hcnt):
        for b in range(NRING):
            t = t4 * NRING + b

            @pl.when(t < myblocks)
            def _(b=b):
                pltpu.make_async_copy(tblT.at[:, pl.ds(0, BW)],
                                      ring.at[b], sems[b]).wait()

            hcnt = scan_one(t, hcnt, b)

            @pl.when(t + NRING < myblocks)
            def _(t=t, b=b):
                fire_block(t + NRING, b)
        return hcnt

    hcnt = lax.fori_loop(0, (CBW + NRING - 1) // NRING, scan_quad, 0)
    lax.fori_loop(0, hcnt & 3, drain_rows, 0)


def _dot_body(urows, mrows, out_hbm, ubuf, mbuf, outv, sem_u, sem_m):
    wid = lax.axis_index("s") * NC + lax.axis_index("c")
    lane = lax.iota(jnp.int32, L)

    def chunk(q, _):
        cb = wid * BPW + q * 128
        cu = pltpu.async_copy(urows.at[pl.ds(cb, 128), :], ubuf, sem_u)
        cm = pltpu.async_copy(mrows.at[pl.ds(cb, 128), :], mbuf, sem_m)
        cu.wait()
        cm.wait()

        def grp(g, _):
            rows = g * L + lane
            def dloop(d, acc):
                dv = jnp.full((L,), d, jnp.int32)
                u = plsc.load_gather(ubuf, [rows, dv])
                m = plsc.load_gather(mbuf, [rows, dv])
                return acc + u * m
            acc = lax.fori_loop(0, EMB_, dloop,
                                jnp.zeros((L,), jnp.float32))
            outv[pl.ds(q * 128 + g * L, L)] = acc
            return 0
        lax.fori_loop(0, 128 // L, grp, 0)
        return 0

    lax.fori_loop(0, BPW // 128, chunk, 0)
    pltpu.sync_copy(outv, out_hbm.at[wid])


@jax.jit
def _mf_dot(user_id, movie_id, user_table, movie_table):
    mesh = plsc.VectorSubcoreMesh(core_axis_name="c", subcore_axis_name="s")
    cp = pltpu.CompilerParams(needs_layout_passes=False)

    extract = pl.kernel(
        _extract_body,
        out_type=jax.ShapeDtypeStruct((NDROWS, 128), jnp.float32),
        mesh=mesh,
        compiler_params=cp,
        scratch_types=[
            pltpu.VMEM((BATCH_,), jnp.int32),
            pltpu.VMEM(((CBW + 1) * L,), jnp.int32),
            pltpu.VMEM((CBW * SLOTS * L,), jnp.int32),
            pltpu.VMEM((CBW * SLOTS * L,), jnp.int32),
            pltpu.VMEM((NRING, EMB_, BW), jnp.float32),
            pltpu.VMEM((4, L, 128), jnp.float32),
            pltpu.SemaphoreType.DMA,
            pltpu.SemaphoreType.DMA,
            pltpu.SemaphoreType.DMA,
            pltpu.SemaphoreType.DMA,
            pltpu.SemaphoreType.DMA,
        ],
    )

    dot = pl.kernel(
        _dot_body,
        out_type=jax.ShapeDtypeStruct((NW, BPW), jnp.float32),
        mesh=mesh,
        compiler_params=cp,
        scratch_types=[
            pltpu.VMEM((128, 128), jnp.float32),
            pltpu.VMEM((128, 128), jnp.float32),
            pltpu.VMEM((BPW,), jnp.float32),
            pltpu.SemaphoreType.DMA,
            pltpu.SemaphoreType.DMA,
        ],
    )

    uid = user_id.astype(jnp.int32)
    mid = movie_id.astype(jnp.int32)
    u_rows = extract(uid, user_table.T)
    m_rows = extract(mid, movie_table.T)
    out = dot(u_rows, m_rows)
    return out.reshape(BATCH_)


def kernel(user_id, movie_id, user_table, movie_table):
    return _mf_dot(user_id, movie_id, user_table, movie_table)


# speedup vs baseline: 1.1692x; 1.1692x over previous
import functools

import jax
import jax.numpy as jnp
from jax import lax
from jax.experimental import pallas as pl
from jax.experimental.pallas import tpu as pltpu
from jax.experimental.pallas import tpu_sc as plsc

NC = 2
NS = 16
L = 16
NW = NC * NS

NROWS = 1000000
BATCH_ = 16384
EMB_ = 64
BPW = BATCH_ // NW
NCOL = (NROWS + 127) // 128
BW = 512
BWS = BW.bit_length() - 1
NRING = 2
NCB = (NCOL * 128 + BW - 1) // BW
CBW = 62
MAXOFF = NCOL * 128 - BW
SLOTS = 12
NDROWS = BATCH_ + NW


def _extract_body(ids_hbm, tblT, rows_out,
                  allids, cnt, cand_id, cand_pos, ring, rowring,
                  sem_s0, sem_s1, sem_w):
    wid = lax.axis_index("s") * NC + lax.axis_index("c")
    base_blk = wid * CBW
    myblocks = jnp.clip(NCB - base_blk, 0, CBW)

    pltpu.sync_copy(ids_hbm, allids)

    lane = lax.iota(jnp.int32, L)
    zeros = jnp.zeros((L,), jnp.int32)
    ones = jnp.full((L,), 1, jnp.int32)

    def zcnt(i, _):
        cnt[pl.ds(i * L, L)] = zeros
        return 0
    lax.fori_loop(0, CBW + 1, zcnt, 0)

    def filt(c, _):
        v = allids[pl.ds(c * L, L)]
        blk = lax.shift_right_logical(v, BWS) - base_blk
        mask = (blk >= 0) & (blk < myblocks)
        b = jnp.clip(blk, 0, CBW - 1)
        key = b * L + lane
        o = jnp.minimum(plsc.load_gather(cnt, [key]), SLOTS - 1)
        flat = (b * SLOTS + o) * L + lane
        plsc.store_scatter(cand_id, [flat], v, mask=mask)
        plsc.store_scatter(cand_pos, [flat], c * L + lane, mask=mask)
        plsc.addupdate_scatter(cnt, [key], ones, mask=mask)
        return 0
    lax.fori_loop(0, BATCH_ // L, filt, 0)

    def blk_off(t):
        return jnp.minimum((base_blk + t) * BW, MAXOFF)

    sems = (sem_s0, sem_s1)

    def fire_block(t, b):
        pltpu.async_copy(tblT.at[:, pl.ds(blk_off(t), BW)],
                         ring.at[b], sems[b])

    for b in range(NRING):
        @pl.when(myblocks > b)
        def _(b=b):
            fire_block(b, b)

    def drain_rows(i, _):
        pltpu.make_async_copy(rowring.at[0], rows_out.at[pl.ds(0, L)],
                              sem_w).wait()
        return 0

    def bcast(vec, k):
        return lax.gather(
            vec, jnp.full((L, 1), k, jnp.int32),
            lax.GatherDimensionNumbers(
                offset_dims=(), collapsed_slice_dims=(0,),
                start_index_map=(0,)),
            (1,), mode=lax.GatherScatterMode.PROMISE_IN_BOUNDS)

    def scan_one(t, hcnt, b):
        cvec = cnt[pl.ds(t * L, L)]
        jmax = jnp.max(cvec)
        slot = jnp.full((L,), b, jnp.int32)
        off = blk_off(t)

        def scan_slot(j, hcnt):
            row = cand_id[pl.ds((t * SLOTS + j) * L, L)]
            pvec = cand_pos[pl.ds((t * SLOTS + j) * L, L)]
            hit = cvec > j
            key = jnp.where(hit, zeros, ones)
            _, ids_s = plsc.sort_key_val(key, row)
            _, pos_s = plsc.sort_key_val(key, pvec)
            nh = jnp.sum(jnp.where(hit, ones, zeros))
            cv_all = jnp.clip(ids_s - off, 0, BW - 1)
            hs = hcnt & 7

            def perhit(k, _):
                ck = bcast(cv_all, k)
                for j4 in range(EMB_ // L):
                    dv = j4 * L + lane
                    vals = plsc.load_gather(ring, [slot, dv, ck])
                    rowring[hs, k, pl.ds(j4 * L, L)] = vals
                return 0
            lax.fori_loop(0, nh, perhit, 0)

            posv = jnp.where(lane < nh, pos_s, BATCH_ + wid)
            pltpu.async_copy(rowring.at[hs], rows_out.at[posv], sem_w)
            @pl.when((hcnt & 7) == 7)
            def _():
                lax.fori_loop(0, 8, drain_rows, 0)

            return hcnt + 1

        return lax.fori_loop(0, jmax, scan_slot, hcnt)

    def scan_quad(t4, hcnt):
        for b in range(NRING):
            t = t4 * NRING + b

            @pl.when(t < myblocks)
            def _(b=b):
                pltpu.make_async_copy(tblT.at[:, pl.ds(0, BW)],
                                      ring.at[b], sems[b]).wait()

            hcnt = scan_one(t, hcnt, b)

            @pl.when(t + NRING < myblocks)
            def _(t=t, b=b):
                fire_block(t + NRING, b)
        return hcnt

    hcnt = lax.fori_loop(0, (CBW + NRING - 1) // NRING, scan_quad, 0)
    lax.fori_loop(0, hcnt & 7, drain_rows, 0)


def _dot_body(urows, mrows, out_hbm, ubuf, mbuf, outv, sem_u, sem_m):
    wid = lax.axis_index("s") * NC + lax.axis_index("c")
    lane = lax.iota(jnp.int32, L)

    def chunk(q, _):
        cb = wid * BPW + q * 128
        cu = pltpu.async_copy(urows.at[pl.ds(cb, 128), :], ubuf, sem_u)
        cm = pltpu.async_copy(mrows.at[pl.ds(cb, 128), :], mbuf, sem_m)
        cu.wait()
        cm.wait()

        def grp(g, _):
            rows = g * L + lane
            def dloop(d, acc):
                dv = jnp.full((L,), d, jnp.int32)
                u = plsc.load_gather(ubuf, [rows, dv])
                m = plsc.load_gather(mbuf, [rows, dv])
                return acc + u * m
            acc = lax.fori_loop(0, EMB_, dloop,
                                jnp.zeros((L,), jnp.float32))
            outv[pl.ds(q * 128 + g * L, L)] = acc
            return 0
        lax.fori_loop(0, 128 // L, grp, 0)
        return 0

    lax.fori_loop(0, BPW // 128, chunk, 0)
    pltpu.sync_copy(outv, out_hbm.at[wid])


@jax.jit
def _mf_dot(user_id, movie_id, user_table, movie_table):
    mesh = plsc.VectorSubcoreMesh(core_axis_name="c", subcore_axis_name="s")
    cp = pltpu.CompilerParams(needs_layout_passes=False)

    extract = pl.kernel(
        _extract_body,
        out_type=jax.ShapeDtypeStruct((NDROWS, 128), jnp.float32),
        mesh=mesh,
        compiler_params=cp,
        scratch_types=[
            pltpu.VMEM((BATCH_,), jnp.int32),
            pltpu.VMEM(((CBW + 1) * L,), jnp.int32),
            pltpu.VMEM((CBW * SLOTS * L,), jnp.int32),
            pltpu.VMEM((CBW * SLOTS * L,), jnp.int32),
            pltpu.VMEM((NRING, EMB_, BW), jnp.float32),
            pltpu.VMEM((8, L, 128), jnp.float32),
            pltpu.SemaphoreType.DMA,
            pltpu.SemaphoreType.DMA,
            pltpu.SemaphoreType.DMA,
        ],
    )

    dot = pl.kernel(
        _dot_body,
        out_type=jax.ShapeDtypeStruct((NW, BPW), jnp.float32),
        mesh=mesh,
        compiler_params=cp,
        scratch_types=[
            pltpu.VMEM((128, 128), jnp.float32),
            pltpu.VMEM((128, 128), jnp.float32),
            pltpu.VMEM((BPW,), jnp.float32),
            pltpu.SemaphoreType.DMA,
            pltpu.SemaphoreType.DMA,
        ],
    )

    uid = user_id.astype(jnp.int32)
    mid = movie_id.astype(jnp.int32)
    u_rows = extract(uid, user_table.T)
    m_rows = extract(mid, movie_table.T)
    out = dot(u_rows, m_rows)
    return out.reshape(BATCH_)


def kernel(user_id, movie_id, user_table, movie_table):
    return _mf_dot(user_id, movie_id, user_table, movie_table)
